# baseline (speedup 1.0000x reference)
import functools

import jax
import jax.numpy as jnp
from jax import lax
from jax.experimental import pallas as pl
from jax.experimental.pallas import tpu as pltpu
from jax.experimental.pallas import tpu_sc as plsc

NX = 512
NY = 512
NCELLS = NX * NY
F = 6
NC = 2
NS = 16
NT = NC * NS
WROWS = 49
CHUNK = WROWS * 128
NP_PAD = NT * CHUNK
ACC = F * NCELLS
ACC_SLICE = ACC // NS
ZB = 4096

_mesh = plsc.VectorSubcoreMesh(core_axis_name="c", subcore_axis_name="s")


@functools.partial(
    pl.kernel,
    mesh=_mesh,
    out_type=jax.ShapeDtypeStruct((NC * ACC,), jnp.float32),
    scratch_types=[
        pltpu.VMEM_SHARED((ACC,), jnp.float32),
        pltpu.VMEM((ZB,), jnp.float32),
        pltpu.VMEM((CHUNK,), jnp.int32),
        pltpu.VMEM((CHUNK,), jnp.int32),
        pltpu.VMEM((CHUNK,), jnp.float32),
        pltpu.VMEM((CHUNK,), jnp.float32),
        pltpu.SemaphoreType.DMA,
        pltpu.SemaphoreType.DMA,
        pltpu.SemaphoreType.DMA,
    ],
)
def _sc_scatter(v0, v1, v2, v3, v4, v5, idx_hbm, part_hbm,
                acc, zb, isha, ishb, vwa, vwb, semz, sla, slb):
    c = lax.axis_index("c")
    s = lax.axis_index("s")
    tile = c * NS + s
    base = tile * CHUNK
    a0 = s * ACC_SLICE
    vfs = (v0, v1, v2, v3, v4, v5)
    bufs = (vwa, vwb)
    ishs = (isha, ishb)
    lsems = (sla, slb)

    idx_load = pltpu.async_copy(idx_hbm.at[pl.ds(base, CHUNK)], isha, slb)
    loads = [pltpu.async_copy(v0.at[pl.ds(base, CHUNK)], vwa, sla)]

    @pl.loop(0, ZB // 16)
    def _(i):
        zb[pl.ds(i * 16, 16)] = jnp.zeros((16,), jnp.float32)

    zcopies = [
        pltpu.async_copy(zb, acc.at[pl.ds(a0 + i * ZB, ZB)], semz)
        for i in range(ACC_SLICE // ZB)
    ]
    for zc in zcopies:
        zc.wait()
    idx_load.wait()
    plsc.subcore_barrier()

    for f in range(F):
        b = f % 2
        if f + 1 < F:
            loads.append(
                pltpu.async_copy(vfs[f + 1].at[pl.ds(base, CHUNK)],
                                 bufs[1 - b], lsems[1 - b]))

            @pl.loop(0, CHUNK // 128)
            def _(i, b=b):
                for j in range(8):
                    sl = pl.ds(i * 128 + j * 16, 16)
                    ishs[1 - b][sl] = ishs[b][sl] + NCELLS

        loads[f].wait()
        pltpu.sync_copy(bufs[b], acc.at[ishs[b]], add=True)

    plsc.subcore_barrier()
    pltpu.sync_copy(acc.at[pl.ds(a0, ACC_SLICE)],
                    part_hbm.at[pl.ds(c * ACC + a0, ACC_SLICE)])


_DGRID = 7
_DB = NP_PAD // _DGRID
_DOROWS = _DB // 128


def _tc_deint_body(x_ref, o0, o1, o2, o3, o4, o5):
    i = pl.program_id(0)
    rem = 200000 - i * _DB
    pos = lax.broadcasted_iota(jnp.int32, (_DB, 1), 0)
    valid = pos < rem
    xb = x_ref[...]
    outs = (o0, o1, o2, o3, o4, o5)
    for f in range(F):
        col = jnp.where(valid, xb[:, f:f + 1], 0.0)
        outs[f][...] = col.reshape(_DOROWS, 128)


_vplane = jax.ShapeDtypeStruct((NP_PAD // 128, 128), jnp.float32)

_tc_deint = pl.pallas_call(
    _tc_deint_body,
    grid=(_DGRID,),
    in_specs=[pl.BlockSpec((_DB, F), lambda i: (i, 0))],
    out_specs=[pl.BlockSpec((_DOROWS, 128), lambda i: (i, 0))] * F,
    out_shape=[_vplane] * F,
)

_TCROWS = NCELLS // 128


def _tc_assemble_body(a_ref, b_ref, o_ref):
    s = a_ref[...] + b_ref[...]
    o_ref[...] = s.reshape(1, NX, NY)


_tc_assemble = pl.pallas_call(
    _tc_assemble_body,
    grid=(F,),
    in_specs=[
        pl.BlockSpec((_TCROWS, 128), lambda i: (i, 0)),
        pl.BlockSpec((_TCROWS, 128), lambda i: (i + F, 0)),
    ],
    out_specs=pl.BlockSpec((1, NX, NY), lambda i: (i, 0, 0)),
    out_shape=jax.ShapeDtypeStruct((F, NX, NY), jnp.float32),
)


def kernel(x, indices):
    n = x.shape[0]
    idx = indices.astype(jnp.int32)
    npad = NP_PAD - n
    idx_pad = jnp.concatenate(
        [idx, (jnp.arange(npad, dtype=jnp.int32) * 97) % NCELLS])
    vfs = [v.reshape(NP_PAD) for v in _tc_deint(x.astype(jnp.float32))]
    part = _sc_scatter(*vfs, idx_pad)
    part2d = part.reshape(NC * F * _TCROWS, 128)
    return _tc_assemble(part2d, part2d)


# speedup vs baseline: 2.6025x; 2.6025x over previous
import functools

import jax
import jax.numpy as jnp
from jax import lax
from jax.experimental import pallas as pl
from jax.experimental.pallas import tpu as pltpu
from jax.experimental.pallas import tpu_sc as plsc

NX = 512
NY = 512
NCELLS = NX * NY
F = 6
NC = 2
FC = F // NC
NS = 16
CHUNK = 12544
NP_PAD = NS * CHUNK
ACC = FC * NCELLS
ACC_SLICE = ACC // NS
ZB = 4096

_mesh = plsc.VectorSubcoreMesh(core_axis_name="c", subcore_axis_name="s")


@functools.partial(
    pl.kernel,
    mesh=_mesh,
    out_type=jax.ShapeDtypeStruct((NC * ACC,), jnp.float32),
    scratch_types=[
        pltpu.VMEM_SHARED((ACC,), jnp.float32),
        pltpu.VMEM((ZB,), jnp.float32),
        pltpu.VMEM((CHUNK,), jnp.int32),
        pltpu.VMEM((CHUNK,), jnp.int32),
        pltpu.VMEM((CHUNK,), jnp.float32),
        pltpu.VMEM((CHUNK,), jnp.float32),
        pltpu.SemaphoreType.DMA,
        pltpu.SemaphoreType.DMA,
        pltpu.SemaphoreType.DMA,
    ],
)
def _sc_scatter(vcat_hbm, idx_hbm, part_hbm,
                acc, zb, isha, ishb, vwa, vwb, semz, sla, slb):
    c = lax.axis_index("c")
    s = lax.axis_index("s")
    base = s * CHUNK
    a0 = s * ACC_SLICE
    bufs = (vwa, vwb)
    ishs = (isha, ishb)
    lsems = (sla, slb)
    f0 = c * FC

    idx_load = pltpu.async_copy(idx_hbm.at[pl.ds(base, CHUNK)], isha, slb)
    loads = [pltpu.async_copy(
        vcat_hbm.at[pl.ds(f0 * NP_PAD + base, CHUNK)], vwa, sla)]

    @pl.loop(0, ZB // 16)
    def _(i):
        zb[pl.ds(i * 16, 16)] = jnp.zeros((16,), jnp.float32)

    zcopies = [
        pltpu.async_copy(zb, acc.at[pl.ds(a0 + i * ZB, ZB)], semz)
        for i in range(ACC_SLICE // ZB)
    ]
    for zc in zcopies:
        zc.wait()
    idx_load.wait()
    plsc.subcore_barrier()

    for f in range(FC):
        b = f % 2
        if f + 1 < FC:
            loads.append(pltpu.async_copy(
                vcat_hbm.at[pl.ds((f0 + f + 1) * NP_PAD + base, CHUNK)],
                bufs[1 - b], lsems[1 - b]))

            @pl.loop(0, CHUNK // 128)
            def _(i, b=b):
                for j in range(8):
                    sl = pl.ds(i * 128 + j * 16, 16)
                    ishs[1 - b][sl] = ishs[b][sl] + NCELLS

        loads[f].wait()
        pltpu.sync_copy(bufs[b], acc.at[ishs[b]], add=True)

    plsc.subcore_barrier()
    pltpu.sync_copy(acc.at[pl.ds(a0, ACC_SLICE)],
                    part_hbm.at[pl.ds(c * ACC + a0, ACC_SLICE)])


_TCROWS = NCELLS // 128


def _tc_assemble_body(a_ref, o_ref):
    o_ref[...] = a_ref[...].reshape(1, NX, NY)


_tc_assemble = pl.pallas_call(
    _tc_assemble_body,
    grid=(F,),
    in_specs=[pl.BlockSpec((_TCROWS, 128), lambda i: (i, 0))],
    out_specs=pl.BlockSpec((1, NX, NY), lambda i: (i, 0, 0)),
    out_shape=jax.ShapeDtypeStruct((F, NX, NY), jnp.float32),
)


def kernel(x, indices):
    n = x.shape[0]
    idx = indices.astype(jnp.int32)
    npad = NP_PAD - n
    idx_pad = jnp.concatenate(
        [idx, (jnp.arange(npad, dtype=jnp.int32) * 97) % NCELLS])
    xf = x.astype(jnp.float32)
    zpad = jnp.zeros((npad,), jnp.float32)
    vcat = jnp.concatenate(
        [jnp.concatenate([xf[:, f], zpad]) for f in range(F)])
    part = _sc_scatter(vcat, idx_pad)
    return _tc_assemble(part.reshape(F * _TCROWS, 128))
